# baseline (speedup 1.0000x reference)
import jax
import jax.numpy as jnp
import numpy as np
from jax.experimental import pallas as pl
from jax.experimental.pallas import tpu as pltpu

_B, _P, _A, _E, _C = 128, 8, 12, 512, 2
_EC = _E * _C
_AC = _A * _C
_S = _A + 1
_BB = 32
_RB = _BB * _P * _S
_G = _BB * _P
_DELTA = 1e-20


def _body(back_ref, back2_ref, fwd_ref, wpack_ref, rp_ref, dm_ref, sseg_ref,
          p24_ref, tcum_ref, perm_ref, out_ref):
    wselt = wpack_ref[0:_AC, :]
    bsel = wpack_ref[_AC:_AC + 1, 0:_AC]
    dots = (((1,), (1,)), ((), ()))

    back_a = back_ref[...].reshape(_RB // 2, _EC)
    back_b = back2_ref[...].reshape(_RB // 2, _EC)
    y_a = jax.lax.dot_general(back_a, wselt, dots,
                              preferred_element_type=jnp.float32)
    y_b = jax.lax.dot_general(back_b, wselt, dots,
                              preferred_element_type=jnp.float32)
    y = jnp.concatenate([y_a, y_b], axis=0) + bsel
    sp = jax.nn.softplus(y)
    masked = sp * dm_ref[...]
    g1 = jnp.dot(sseg_ref[...], masked,
                 preferred_element_type=jnp.float32)
    f_in = jnp.dot(g1, p24_ref[...],
                   preferred_element_type=jnp.float32)

    fwd = fwd_ref[...].reshape(_G, _EC)
    yf = jax.lax.dot_general(fwd, wselt, dots,
                             preferred_element_type=jnp.float32) + bsel
    f_out = jnp.dot(jax.nn.softplus(yf), p24_ref[...],
                    preferred_element_type=jnp.float32)

    rew = rp_ref[:, 0:2]
    pif = rp_ref[:, 2:4]
    f_init = pif * jnp.exp(rp_ref[:, 4:6])

    logterm = jnp.log(_DELTA + f_out) - jnp.log(_DELTA + f_out + rew)
    p_out = jnp.dot(tcum_ref[...], logterm,
                    preferred_element_type=jnp.float32)

    cat = jnp.concatenate([f_in, f_out, rew, f_init, p_out, rew], axis=1)
    out_ref[...] = jnp.dot(cat, perm_ref[...],
                           preferred_element_type=jnp.float32)


def kernel(forward_edges, backward_edges, path_init_flow, paths_reward, W, b,
           initial_flow):
    back3 = (backward_edges.reshape(_B, _P, _S, 4, 128, _C)
             .swapaxes(4, 5).reshape(_B * _P * _S, 8, 128))
    fwd4 = (forward_edges.reshape(_B, _P, _S, 4, 128, _C)
            .swapaxes(4, 5).reshape(_B, _P, _S * 8, 128))
    wlin = (W.reshape(_C, 4, 128, _A).transpose(3, 1, 0, 2)
            .reshape(_A, _EC))

    posc = (np.arange(_EC) // 128) % _C
    jj = np.arange(_AC)
    colmask = jnp.asarray((posc[None, :] == (jj[:, None] % _C))
                          .astype(np.float32))
    wselt = jnp.repeat(wlin, _C, axis=0) * colmask
    brow = jnp.concatenate(
        [jnp.transpose(b).reshape(1, _AC),
         jnp.zeros((1, _EC - _AC), jnp.float32)], axis=1)
    wpack = jnp.concatenate([wselt, brow], axis=0)

    rp = jnp.concatenate(
        [paths_reward.reshape(_B * _P, _C),
         path_init_flow.reshape(_B * _P, _C),
         jnp.broadcast_to(initial_flow, (_B * _P, _C))], axis=1)

    r = np.arange(_RB)
    s = r % _S
    dm = ((s[:, None] >= 1) & (jj[None, :] // _C == s[:, None] - 1)
          ).astype(np.float32)
    sseg = (r[None, :] // _S == np.arange(_G)[:, None]).astype(np.float32)
    p24 = (jj[:, None] % _C == np.arange(_C)[None, :]).astype(np.float32)
    g = np.arange(_G)
    tcum = ((g[:, None] // _P == g[None, :] // _P)
            & (g[None, :] % _P < g[:, None] % _P)).astype(np.float32)
    kk = np.arange(12)
    perm = ((kk[:, None] % _C) * 6 + kk[:, None] // _C == kk[None, :]
            ).astype(np.float32)

    grid = (_B // _BB,)
    out = pl.pallas_call(
        _body,
        grid=grid,
        in_specs=[
            pl.BlockSpec((_RB // 2, 8, 128), lambda i: (2 * i, 0, 0)),
            pl.BlockSpec((_RB // 2, 8, 128), lambda i: (2 * i + 1, 0, 0)),
            pl.BlockSpec((_BB, _P, 8, 128), lambda i: (i, 0, 0, 0)),
            pl.BlockSpec((_AC + 1, _EC), lambda i: (0, 0)),
            pl.BlockSpec((_G, 6), lambda i: (i, 0)),
            pl.BlockSpec((_RB, _AC), lambda i: (0, 0)),
            pl.BlockSpec((_G, _RB), lambda i: (0, 0)),
            pl.BlockSpec((_AC, _C), lambda i: (0, 0)),
            pl.BlockSpec((_G, _G), lambda i: (0, 0)),
            pl.BlockSpec((12, 12), lambda i: (0, 0)),
        ],
        out_specs=pl.BlockSpec((_G, 12), lambda i: (i, 0)),
        out_shape=jax.ShapeDtypeStruct((_B * _P, 12), jnp.float32),
        compiler_params=pltpu.CompilerParams(
            dimension_semantics=("arbitrary",)),
    )(back3, back3, fwd4, wpack, rp, jnp.asarray(dm), jnp.asarray(sseg),
      jnp.asarray(p24), jnp.asarray(tcum), jnp.asarray(perm))
    return out.reshape(_B, _P, _C, 6)


# speedup vs baseline: 1.1227x; 1.1227x over previous
import jax
import jax.numpy as jnp
import numpy as np
from jax.experimental import pallas as pl
from jax.experimental.pallas import tpu as pltpu

_B, _P, _A, _E, _C = 128, 8, 12, 512, 2
_EC = _E * _C
_AC = _A * _C
_S = _A + 1
_BB = 32
_RB = _BB * _P * _S
_G = _BB * _P
_DELTA = 1e-20


def _body(back_ref, fwd_ref, wpack_ref, rp_ref, dm_ref, sseg_ref, p24_ref,
          tcum_ref, perm_ref, out_ref):
    wselt = wpack_ref[0:_AC, :]
    bsel = wpack_ref[_AC:_AC + 1, 0:_AC]
    dots = (((1,), (1,)), ((), ()))

    back = back_ref[...].reshape(_RB, _EC)
    y = jax.lax.dot_general(back, wselt, dots,
                            preferred_element_type=jnp.float32) + bsel
    sp = jax.nn.softplus(y)
    masked = sp * dm_ref[...]
    g1 = jnp.dot(sseg_ref[...], masked,
                 preferred_element_type=jnp.float32)
    f_in = jnp.dot(g1, p24_ref[...],
                   preferred_element_type=jnp.float32)

    fwd = fwd_ref[...].reshape(_G, _EC)
    yf = jax.lax.dot_general(fwd, wselt, dots,
                             preferred_element_type=jnp.float32) + bsel
    f_out = jnp.dot(jax.nn.softplus(yf), p24_ref[...],
                    preferred_element_type=jnp.float32)

    rew = rp_ref[:, 0:2]
    pif = rp_ref[:, 2:4]
    f_init = pif * jnp.exp(rp_ref[:, 4:6])

    logterm = jnp.log(_DELTA + f_out) - jnp.log(_DELTA + f_out + rew)
    p_out = jnp.dot(tcum_ref[...], logterm,
                    preferred_element_type=jnp.float32)

    cat = jnp.concatenate([f_in, f_out, rew, f_init, p_out, rew], axis=1)
    out_ref[...] = jnp.dot(cat, perm_ref[...],
                           preferred_element_type=jnp.float32)


def kernel(forward_edges, backward_edges, path_init_flow, paths_reward, W, b,
           initial_flow):
    back3 = (backward_edges.reshape(_B, _P, _S, 4, 128, _C)
             .swapaxes(4, 5).reshape(_B * _P * _S, 8, 128))
    fwd4 = (forward_edges.reshape(_B, _P, _S, 4, 128, _C)
            .swapaxes(4, 5).reshape(_B, _P, _S * 8, 128))
    wlin = (W.reshape(_C, 4, 128, _A).transpose(3, 1, 0, 2)
            .reshape(_A, _EC))

    posc = (np.arange(_EC) // 128) % _C
    jj = np.arange(_AC)
    colmask = jnp.asarray((posc[None, :] == (jj[:, None] % _C))
                          .astype(np.float32))
    wselt = jnp.repeat(wlin, _C, axis=0) * colmask
    brow = jnp.concatenate(
        [jnp.transpose(b).reshape(1, _AC),
         jnp.zeros((1, _EC - _AC), jnp.float32)], axis=1)
    wpack = jnp.concatenate([wselt, brow], axis=0)

    rp = jnp.concatenate(
        [paths_reward.reshape(_B * _P, _C),
         path_init_flow.reshape(_B * _P, _C),
         jnp.broadcast_to(initial_flow, (_B * _P, _C))], axis=1)

    r = np.arange(_RB)
    s = r % _S
    dm = ((s[:, None] >= 1) & (jj[None, :] // _C == s[:, None] - 1)
          ).astype(np.float32)
    sseg = (r[None, :] // _S == np.arange(_G)[:, None]).astype(np.float32)
    p24 = (jj[:, None] % _C == np.arange(_C)[None, :]).astype(np.float32)
    g = np.arange(_G)
    tcum = ((g[:, None] // _P == g[None, :] // _P)
            & (g[None, :] % _P < g[:, None] % _P)).astype(np.float32)
    kk = np.arange(12)
    perm = ((kk[:, None] % _C) * 6 + kk[:, None] // _C == kk[None, :]
            ).astype(np.float32)

    grid = (_B // _BB,)
    out = pl.pallas_call(
        _body,
        grid=grid,
        in_specs=[
            pl.BlockSpec((_RB, 8, 128), lambda i: (i, 0, 0)),
            pl.BlockSpec((_BB, _P, 8, 128), lambda i: (i, 0, 0, 0)),
            pl.BlockSpec((_AC + 1, _EC), lambda i: (0, 0)),
            pl.BlockSpec((_G, 6), lambda i: (i, 0)),
            pl.BlockSpec((_RB, _AC), lambda i: (0, 0)),
            pl.BlockSpec((_G, _RB), lambda i: (0, 0)),
            pl.BlockSpec((_AC, _C), lambda i: (0, 0)),
            pl.BlockSpec((_G, _G), lambda i: (0, 0)),
            pl.BlockSpec((12, 12), lambda i: (0, 0)),
        ],
        out_specs=pl.BlockSpec((_G, 12), lambda i: (i, 0)),
        out_shape=jax.ShapeDtypeStruct((_B * _P, 12), jnp.float32),
        compiler_params=pltpu.CompilerParams(
            dimension_semantics=("arbitrary",)),
    )(back3, fwd4, wpack, rp, jnp.asarray(dm), jnp.asarray(sseg),
      jnp.asarray(p24), jnp.asarray(tcum), jnp.asarray(perm))
    return out.reshape(_B, _P, _C, 6)
